# Initial kernel scaffold; baseline (speedup 1.0000x reference)
import jax
import jax.numpy as jnp
from jax.experimental import pallas as pl


def kernel(context, gate_w, gate_b, w1, b1, w2, b2):
    raise NotImplementedError("write your pallas kernel here")


# speedup vs baseline: 5.2295x; 5.2295x over previous
import functools
import math

import jax
import jax.numpy as jnp
from jax import lax
from jax.experimental import pallas as pl
from jax.experimental.pallas import tpu as pltpu
from jax.experimental.pallas import tpu_sc as plsc

_E = 64
_TOPK = 2
_TEMP = 1.0
_CAPF = 1.1

_TB = 256
_RB = 128


def _gating_body(x_ref, gw_ref, gb_ref,
                 i1_ref, i2_ref, s1_ref, s2_ref, imp_ref, ent_ref, cnt_ref):
    pid = pl.program_id(0)
    x = x_ref[...]
    gw = gw_ref[...]
    gb = gb_ref[...]
    logits = lax.dot_general(x, gw, (((1,), (1,)), ((), ())),
                             preferred_element_type=jnp.float32) + gb
    scaled = logits / _TEMP
    m = jnp.max(scaled, axis=1, keepdims=True)
    p = jnp.exp(scaled - m)
    probs = p / jnp.sum(p, axis=1, keepdims=True)
    ent = -jnp.sum(probs * jnp.log(jnp.clip(probs, 1e-9, None)))
    cols = lax.broadcasted_iota(jnp.int32, scaled.shape, 1)
    v1 = jnp.max(scaled, axis=1)
    i1 = jnp.min(jnp.where(scaled == v1[:, None], cols, _E), axis=1)
    masked = jnp.where(cols == i1[:, None], -jnp.inf, scaled)
    v2 = jnp.max(masked, axis=1)
    i2 = jnp.min(jnp.where(masked == v2[:, None], cols, _E), axis=1)
    t = jnp.exp(v2 - v1)
    s1 = 1.0 / (1.0 + t)
    s2 = t / (1.0 + t)
    i1_ref[0, 0, :] = i1
    i2_ref[0, 0, :] = i2
    s1_ref[0, 0, :] = s1
    s2_ref[0, 0, :] = s2

    oh = ((cols == i1[:, None]).astype(jnp.float32)
          + (cols == i2[:, None]).astype(jnp.float32))

    @pl.when(pid == 0)
    def _():
        imp_ref[...] = jnp.zeros_like(imp_ref)
        ent_ref[...] = jnp.zeros_like(ent_ref)
        cnt_ref[...] = jnp.zeros_like(cnt_ref)

    imp_ref[...] += jnp.sum(probs, axis=0, keepdims=True)
    ent_ref[...] += jnp.full((1, 1), 0.0) + ent
    cnt_ref[...] += jnp.sum(oh, axis=0, keepdims=True)


def _gating(context, gate_w, gate_b):
    n, dim = context.shape
    nb = n // _TB
    out_shapes = [
        jax.ShapeDtypeStruct((nb, 1, _TB), jnp.int32),
        jax.ShapeDtypeStruct((nb, 1, _TB), jnp.int32),
        jax.ShapeDtypeStruct((nb, 1, _TB), jnp.float32),
        jax.ShapeDtypeStruct((nb, 1, _TB), jnp.float32),
        jax.ShapeDtypeStruct((1, _E), jnp.float32),
        jax.ShapeDtypeStruct((1, 1), jnp.float32),
        jax.ShapeDtypeStruct((1, _E), jnp.float32),
    ]
    tok_spec = pl.BlockSpec((1, 1, _TB), lambda i: (i, 0, 0))
    acc_e = pl.BlockSpec((1, _E), lambda i: (0, 0))
    return pl.pallas_call(
        _gating_body,
        grid=(nb,),
        in_specs=[
            pl.BlockSpec((_TB, dim), lambda i: (i, 0)),
            pl.BlockSpec((_E, dim), lambda i: (0, 0)),
            pl.BlockSpec((1, _E), lambda i: (0, 0)),
        ],
        out_specs=[tok_spec, tok_spec, tok_spec, tok_spec,
                   acc_e, pl.BlockSpec((1, 1), lambda i: (0, 0)), acc_e],
        out_shape=out_shapes,
    )(context, gate_w, gate_b.reshape(1, _E))


def _rank_body(cap, cap_pad, sent, ec_ref, sc_ref, er_ref, sr_ref,
               dst_ref, wgt_ref):
    b = pl.program_id(0)
    ec = ec_ref[...]
    sc = sc_ref[...]
    er = er_ref[...]
    sr = sr_ref[...]
    s_total = er.shape[1]
    i_idx = b * _RB + lax.broadcasted_iota(jnp.int32, (_RB, 1), 0)
    j_idx = lax.broadcasted_iota(jnp.int32, (1, s_total), 1)
    same = er == ec
    better = (sr > sc) | ((sr == sc) & (j_idx < i_idx))
    rank = jnp.sum((same & better).astype(jnp.int32), axis=1, keepdims=True)
    keep = rank < cap
    dst_ref[...] = jnp.where(keep, ec * cap_pad + rank, sent)
    wgt_ref[...] = jnp.where(keep, sc, 0.0)


def _rank(e_flat, s_flat, cap, cap_pad, sent):
    s_total = e_flat.shape[0]
    nb = s_total // _RB
    return pl.pallas_call(
        functools.partial(_rank_body, cap, cap_pad, sent),
        grid=(nb,),
        in_specs=[
            pl.BlockSpec((_RB, 1), lambda i: (i, 0)),
            pl.BlockSpec((_RB, 1), lambda i: (i, 0)),
            pl.BlockSpec((1, s_total), lambda i: (0, 0)),
            pl.BlockSpec((1, s_total), lambda i: (0, 0)),
        ],
        out_specs=[pl.BlockSpec((_RB, 1), lambda i: (i, 0)),
                   pl.BlockSpec((_RB, 1), lambda i: (i, 0))],
        out_shape=[jax.ShapeDtypeStruct((s_total, 1), jnp.int32),
                   jax.ShapeDtypeStruct((s_total, 1), jnp.float32)],
    )(e_flat.reshape(s_total, 1), s_flat.reshape(s_total, 1),
      e_flat.reshape(1, s_total), s_flat.reshape(1, s_total))


def _make_dispatch_kernel(s_total, rows_total, dim, n_workers):
    rows_per_w = rows_total // n_workers
    chunk = 48
    n_chunks = rows_per_w // chunk
    nc = 2
    mesh = plsc.VectorSubcoreMesh(core_axis_name="c", subcore_axis_name="s")

    @functools.partial(
        pl.kernel, mesh=mesh,
        compiler_params=pltpu.CompilerParams(needs_layout_passes=False),
        out_type=[jax.ShapeDtypeStruct((rows_total, dim), jnp.float32),
                  jax.ShapeDtypeStruct((rows_total,), jnp.float32)],
        scratch_types=[
            pltpu.VMEM((s_total,), jnp.int32),
            pltpu.VMEM((s_total,), jnp.float32),
            pltpu.VMEM((rows_per_w,), jnp.int32),
            pltpu.VMEM((rows_per_w,), jnp.float32),
            pltpu.VMEM((chunk,), jnp.int32),
            pltpu.VMEM((chunk, dim), jnp.float32),
            pltpu.SemaphoreType.DMA,
        ],
    )
    def dispatch_kernel(dst_hbm, wgtf_hbm, ctx_hbm, x_hbm, wgtb_hbm,
                        dst_v, wgtf_v, tok_v, wgtb_v, idx_v, rows_v, sem):
        wid = lax.axis_index("s") * nc + lax.axis_index("c")
        base = wid * rows_per_w
        pltpu.sync_copy(dst_hbm, dst_v)
        pltpu.sync_copy(wgtf_hbm, wgtf_v)
        zero = wid * 0
        zi = lax.broadcast(zero, (16,))
        zf = lax.broadcast(zero.astype(jnp.float32), (16,))
        for k in range(rows_per_w // 16):
            tok_v[pl.ds(k * 16, 16)] = zi
            wgtb_v[pl.ds(k * 16, 16)] = zf

        def scan_body(i, _):
            d16 = dst_v[pl.ds(i * 16, 16)]
            w16 = wgtf_v[pl.ds(i * 16, 16)]
            slot = i * 16 + lax.iota(jnp.int32, 16)
            tok16 = lax.shift_right_logical(slot, 1)
            msk = (d16 >= base) & (d16 < base + rows_per_w)
            loc = jnp.where(msk, d16 - base, 0)
            plsc.store_scatter(tok_v, [loc], tok16, mask=msk)
            plsc.store_scatter(wgtb_v, [loc], w16, mask=msk)
            return 0

        lax.fori_loop(0, s_total // 16, scan_body, 0)
        pltpu.sync_copy(wgtb_v, wgtb_hbm.at[pl.ds(base, rows_per_w)])
        for c in range(n_chunks):
            for k in range(chunk // 16):
                idx_v[pl.ds(k * 16, 16)] = tok_v[pl.ds(c * chunk + k * 16, 16)]
            pltpu.async_copy(ctx_hbm.at[idx_v], rows_v, sem).wait()
            pltpu.sync_copy(rows_v, x_hbm.at[pl.ds(base + c * chunk, chunk)])

    return dispatch_kernel


def _ffn_body(x_ref, w1_ref, b1_ref, w2_ref, b2_ref, wgt_ref, y_ref):
    x = x_ref[0]
    w1 = w1_ref[0]
    w2 = w2_ref[0]
    h = lax.dot_general(x, w1, (((1,), (1,)), ((), ())),
                        preferred_element_type=jnp.float32) + b1_ref[0]
    h = jnp.maximum(h, 0.0)
    y = lax.dot_general(h, w2, (((1,), (1,)), ((), ())),
                        preferred_element_type=jnp.float32) + b2_ref[0]
    y_ref[0] = y * wgt_ref[0]


def _ffn(x, w1, b1, w2, b2, wgt_buf, cap_pad):
    e, hidden, dim = w1.shape
    return pl.pallas_call(
        _ffn_body,
        grid=(e,),
        in_specs=[
            pl.BlockSpec((1, cap_pad, dim), lambda i: (i, 0, 0)),
            pl.BlockSpec((1, hidden, dim), lambda i: (i, 0, 0)),
            pl.BlockSpec((1, 1, hidden), lambda i: (i, 0, 0)),
            pl.BlockSpec((1, dim, hidden), lambda i: (i, 0, 0)),
            pl.BlockSpec((1, 1, dim), lambda i: (i, 0, 0)),
            pl.BlockSpec((1, cap_pad, 1), lambda i: (i, 0, 0)),
        ],
        out_specs=pl.BlockSpec((1, cap_pad, dim), lambda i: (i, 0, 0)),
        out_shape=jax.ShapeDtypeStruct((e, cap_pad, dim), jnp.float32),
    )(x.reshape(e, cap_pad, dim), w1, b1.reshape(e, 1, hidden),
      w2, b2.reshape(e, 1, dim), wgt_buf.reshape(e, cap_pad, 1))


def _make_combine_kernel(n, dim, n_workers):
    tok_per_w = n // n_workers
    chunk = 16
    n_chunks = tok_per_w // chunk
    nc = 2
    mesh = plsc.VectorSubcoreMesh(core_axis_name="c", subcore_axis_name="s")

    @functools.partial(
        pl.kernel, mesh=mesh,
        out_type=jax.ShapeDtypeStruct((n, dim), jnp.float32),
        scratch_types=[
            pltpu.VMEM((chunk,), jnp.int32),
            pltpu.VMEM((chunk,), jnp.int32),
            pltpu.VMEM((chunk, dim), jnp.float32),
            pltpu.VMEM((chunk, dim), jnp.float32),
            pltpu.VMEM((chunk, dim), jnp.float32),
            pltpu.SemaphoreType.DMA,
            pltpu.SemaphoreType.DMA,
        ],
    )
    def combine_kernel(y_hbm, g0_hbm, g1_hbm, out_hbm,
                       i0_v, i1_v, a_v, b_v, o_v, sem_a, sem_b):
        wid = lax.axis_index("s") * nc + lax.axis_index("c")
        tbase = wid * tok_per_w
        for c in range(n_chunks):
            t0 = tbase + c * chunk
            pltpu.sync_copy(g0_hbm.at[pl.ds(t0, chunk)], i0_v)
            pltpu.sync_copy(g1_hbm.at[pl.ds(t0, chunk)], i1_v)
            cp_a = pltpu.async_copy(y_hbm.at[i0_v], a_v, sem_a)
            cp_b = pltpu.async_copy(y_hbm.at[i1_v], b_v, sem_b)
            cp_a.wait()
            cp_b.wait()

            def add_row(t, _):
                def add_vec(k, _2):
                    s = pl.ds(k * 16, 16)
                    o_v[t, s] = a_v[t, s] + b_v[t, s]
                    return 0
                lax.fori_loop(0, dim // 16, add_vec, 0)
                return 0

            lax.fori_loop(0, chunk, add_row, 0)
            pltpu.sync_copy(o_v, out_hbm.at[pl.ds(t0, chunk)])

    return combine_kernel


def kernel(context, gate_w, gate_b, w1, b1, w2, b2):
    n, dim = context.shape
    e, hidden, _ = w1.shape
    cap = max(1, math.ceil(n * _TOPK / float(e) * _CAPF))
    cap_pad = cap + 1
    rows_total = e * cap_pad
    sent = rows_total - 1
    s_total = n * _TOPK
    n_workers = 32

    i1, i2, s1, s2, imp_sum, ent_sum, cnt = _gating(context, gate_w, gate_b)
    i1 = i1.reshape(n)
    i2 = i2.reshape(n)
    s1 = s1.reshape(n)
    s2 = s2.reshape(n)
    e_flat = jnp.stack([i1, i2], axis=-1).reshape(-1)
    s_flat = jnp.stack([s1, s2], axis=-1).reshape(-1)

    dst, wgt_flat = _rank(e_flat, s_flat, cap, cap_pad, sent)
    dst = dst.reshape(-1)
    wgt_flat = wgt_flat.reshape(-1)

    x_buf, wgt_buf = _make_dispatch_kernel(s_total, rows_total, dim, n_workers)(
        dst, wgt_flat, context)

    y = _ffn(x_buf, w1, b1, w2, b2, wgt_buf, cap_pad)
    y = y.reshape(rows_total, dim)

    g = dst.reshape(n, _TOPK)
    output = _make_combine_kernel(n, dim, n_workers)(
        y, g[:, 0], g[:, 1])

    dispatch = jnp.minimum(cnt.reshape(e), float(cap))
    load = dispatch / jnp.maximum(dispatch.sum(), 1.0)
    importance = imp_sum.reshape(e) / n
    aux_loss = (importance * load).sum() * e
    entropy = ent_sum.reshape(()) / n
    return output, aux_loss, entropy


# speedup vs baseline: 5.4575x; 1.0436x over previous
import functools
import math

import jax
import jax.numpy as jnp
from jax import lax
from jax.experimental import pallas as pl
from jax.experimental.pallas import tpu as pltpu
from jax.experimental.pallas import tpu_sc as plsc

_E = 64
_TOPK = 2
_TEMP = 1.0
_CAPF = 1.1

_TB = 256
_RB = 128


def _gating_body(x_ref, gw_ref, gb_ref,
                 i1_ref, i2_ref, s1_ref, s2_ref, imp_ref, ent_ref, cnt_ref):
    pid = pl.program_id(0)
    x = x_ref[...]
    gw = gw_ref[...]
    gb = gb_ref[...]
    logits = lax.dot_general(x, gw, (((1,), (1,)), ((), ())),
                             preferred_element_type=jnp.float32) + gb
    scaled = logits / _TEMP
    m = jnp.max(scaled, axis=1, keepdims=True)
    p = jnp.exp(scaled - m)
    probs = p / jnp.sum(p, axis=1, keepdims=True)
    ent = -jnp.sum(probs * jnp.log(jnp.clip(probs, 1e-9, None)))
    cols = lax.broadcasted_iota(jnp.int32, scaled.shape, 1)
    v1 = jnp.max(scaled, axis=1)
    i1 = jnp.min(jnp.where(scaled == v1[:, None], cols, _E), axis=1)
    masked = jnp.where(cols == i1[:, None], -jnp.inf, scaled)
    v2 = jnp.max(masked, axis=1)
    i2 = jnp.min(jnp.where(masked == v2[:, None], cols, _E), axis=1)
    t = jnp.exp(v2 - v1)
    s1 = 1.0 / (1.0 + t)
    s2 = t / (1.0 + t)
    i1_ref[0, 0, :] = i1
    i2_ref[0, 0, :] = i2
    s1_ref[0, 0, :] = s1
    s2_ref[0, 0, :] = s2

    oh = ((cols == i1[:, None]).astype(jnp.float32)
          + (cols == i2[:, None]).astype(jnp.float32))

    @pl.when(pid == 0)
    def _():
        imp_ref[...] = jnp.zeros_like(imp_ref)
        ent_ref[...] = jnp.zeros_like(ent_ref)
        cnt_ref[...] = jnp.zeros_like(cnt_ref)

    imp_ref[...] += jnp.sum(probs, axis=0, keepdims=True)
    ent_ref[...] += jnp.full((1, 1), 0.0) + ent
    cnt_ref[...] += jnp.sum(oh, axis=0, keepdims=True)


def _gating(context, gate_w, gate_b):
    n, dim = context.shape
    nb = n // _TB
    out_shapes = [
        jax.ShapeDtypeStruct((nb, 1, _TB), jnp.int32),
        jax.ShapeDtypeStruct((nb, 1, _TB), jnp.int32),
        jax.ShapeDtypeStruct((nb, 1, _TB), jnp.float32),
        jax.ShapeDtypeStruct((nb, 1, _TB), jnp.float32),
        jax.ShapeDtypeStruct((1, _E), jnp.float32),
        jax.ShapeDtypeStruct((1, 1), jnp.float32),
        jax.ShapeDtypeStruct((1, _E), jnp.float32),
    ]
    tok_spec = pl.BlockSpec((1, 1, _TB), lambda i: (i, 0, 0))
    acc_e = pl.BlockSpec((1, _E), lambda i: (0, 0))
    return pl.pallas_call(
        _gating_body,
        grid=(nb,),
        in_specs=[
            pl.BlockSpec((_TB, dim), lambda i: (i, 0)),
            pl.BlockSpec((_E, dim), lambda i: (0, 0)),
            pl.BlockSpec((1, _E), lambda i: (0, 0)),
        ],
        out_specs=[tok_spec, tok_spec, tok_spec, tok_spec,
                   acc_e, pl.BlockSpec((1, 1), lambda i: (0, 0)), acc_e],
        out_shape=out_shapes,
    )(context, gate_w, gate_b.reshape(1, _E))


def _rank_body(cap, cap_pad, sent, ec_ref, sc_ref, er_ref, sr_ref,
               dst_ref, wgt_ref):
    b = pl.program_id(0)
    ec = ec_ref[...]
    sc = sc_ref[...]
    er = er_ref[...]
    sr = sr_ref[...]
    s_total = er.shape[1]
    i_idx = b * _RB + lax.broadcasted_iota(jnp.int32, (_RB, 1), 0)
    j_idx = lax.broadcasted_iota(jnp.int32, (1, s_total), 1)
    same = er == ec
    better = (sr > sc) | ((sr == sc) & (j_idx < i_idx))
    rank = jnp.sum((same & better).astype(jnp.int32), axis=1, keepdims=True)
    keep = rank < cap
    dst_ref[...] = jnp.where(keep, ec * cap_pad + rank, sent)
    wgt_ref[...] = jnp.where(keep, sc, 0.0)


def _rank(e_flat, s_flat, cap, cap_pad, sent):
    s_total = e_flat.shape[0]
    nb = s_total // _RB
    return pl.pallas_call(
        functools.partial(_rank_body, cap, cap_pad, sent),
        grid=(nb,),
        in_specs=[
            pl.BlockSpec((_RB, 1), lambda i: (i, 0)),
            pl.BlockSpec((_RB, 1), lambda i: (i, 0)),
            pl.BlockSpec((1, s_total), lambda i: (0, 0)),
            pl.BlockSpec((1, s_total), lambda i: (0, 0)),
        ],
        out_specs=[pl.BlockSpec((_RB, 1), lambda i: (i, 0)),
                   pl.BlockSpec((_RB, 1), lambda i: (i, 0))],
        out_shape=[jax.ShapeDtypeStruct((s_total, 1), jnp.int32),
                   jax.ShapeDtypeStruct((s_total, 1), jnp.float32)],
    )(e_flat.reshape(s_total, 1), s_flat.reshape(s_total, 1),
      e_flat.reshape(1, s_total), s_flat.reshape(1, s_total))


def _make_dispatch_kernel(s_total, rows_total, dim, n_workers):
    rows_per_w = rows_total // n_workers
    chunk = 48
    n_chunks = rows_per_w // chunk
    nc = 2
    mesh = plsc.VectorSubcoreMesh(core_axis_name="c", subcore_axis_name="s")

    @functools.partial(
        pl.kernel, mesh=mesh,
        compiler_params=pltpu.CompilerParams(needs_layout_passes=False),
        out_type=[jax.ShapeDtypeStruct((rows_total, dim), jnp.float32),
                  jax.ShapeDtypeStruct((rows_total,), jnp.float32)],
        scratch_types=[
            pltpu.VMEM((s_total,), jnp.int32),
            pltpu.VMEM((s_total,), jnp.float32),
            pltpu.VMEM((rows_per_w,), jnp.int32),
            pltpu.VMEM((rows_per_w,), jnp.float32),
            pltpu.VMEM((chunk,), jnp.int32),
            pltpu.VMEM((chunk,), jnp.int32),
            pltpu.VMEM((chunk, dim), jnp.float32),
            pltpu.VMEM((chunk, dim), jnp.float32),
            pltpu.SemaphoreType.DMA,
            pltpu.SemaphoreType.DMA,
            pltpu.SemaphoreType.DMA,
            pltpu.SemaphoreType.DMA,
        ],
    )
    def dispatch_kernel(dst_hbm, wgtf_hbm, ctx_hbm, x_hbm, wgtb_hbm,
                        dst_v, wgtf_v, tok_v, wgtb_v, idx0_v, idx1_v,
                        rows0_v, rows1_v, sem_g0, sem_g1, sem_w0, sem_w1):
        wid = lax.axis_index("s") * nc + lax.axis_index("c")
        base = wid * rows_per_w
        pltpu.sync_copy(dst_hbm, dst_v)
        pltpu.sync_copy(wgtf_hbm, wgtf_v)
        zero = wid * 0
        zi = lax.broadcast(zero, (16,))
        zf = lax.broadcast(zero.astype(jnp.float32), (16,))
        for k in range(rows_per_w // 16):
            tok_v[pl.ds(k * 16, 16)] = zi
            wgtb_v[pl.ds(k * 16, 16)] = zf

        base_iota = lax.iota(jnp.int32, 16)
        unroll = 4

        def scan_body(i, _):
            for u in range(unroll):
                off = i * (16 * unroll) + u * 16
                d16 = dst_v[pl.ds(off, 16)]
                w16 = wgtf_v[pl.ds(off, 16)]
                slot = off + base_iota
                tok16 = lax.shift_right_logical(slot, 1)
                msk = (d16 >= base) & (d16 < base + rows_per_w)
                loc = jnp.where(msk, d16 - base, 0)
                plsc.store_scatter(tok_v, [loc], tok16, mask=msk)
                plsc.store_scatter(wgtb_v, [loc], w16, mask=msk)
            return 0

        lax.fori_loop(0, s_total // (16 * unroll), scan_body, 0)
        pltpu.async_copy(wgtb_v, wgtb_hbm.at[pl.ds(base, rows_per_w)], sem_w0).wait()

        idx_bufs = (idx0_v, idx1_v)
        row_bufs = (rows0_v, rows1_v)
        g_sems = (sem_g0, sem_g1)
        w_sems = (sem_w0, sem_w1)

        def issue_gather(c):
            idx_v = idx_bufs[c % 2]
            for k in range(chunk // 16):
                idx_v[pl.ds(k * 16, 16)] = tok_v[pl.ds(c * chunk + k * 16, 16)]
            return pltpu.async_copy(ctx_hbm.at[idx_v], row_bufs[c % 2],
                                    g_sems[c % 2])

        g = issue_gather(0)
        w_prev = None
        for c in range(n_chunks):
            g.wait()
            if w_prev is not None:
                w_prev.wait()
            if c + 1 < n_chunks:
                g = issue_gather(c + 1)
            w_prev = pltpu.async_copy(
                row_bufs[c % 2], x_hbm.at[pl.ds(base + c * chunk, chunk)],
                w_sems[c % 2])
        w_prev.wait()

    return dispatch_kernel


def _ffn_body(x_ref, w1_ref, b1_ref, w2_ref, b2_ref, wgt_ref, y_ref):
    x = x_ref[0]
    w1 = w1_ref[0]
    w2 = w2_ref[0]
    h = lax.dot_general(x, w1, (((1,), (1,)), ((), ())),
                        preferred_element_type=jnp.float32) + b1_ref[0]
    h = jnp.maximum(h, 0.0)
    y = lax.dot_general(h, w2, (((1,), (1,)), ((), ())),
                        preferred_element_type=jnp.float32) + b2_ref[0]
    y_ref[0] = y * wgt_ref[0]


def _ffn(x, w1, b1, w2, b2, wgt_buf, cap_pad):
    e, hidden, dim = w1.shape
    return pl.pallas_call(
        _ffn_body,
        grid=(e,),
        in_specs=[
            pl.BlockSpec((1, cap_pad, dim), lambda i: (i, 0, 0)),
            pl.BlockSpec((1, hidden, dim), lambda i: (i, 0, 0)),
            pl.BlockSpec((1, 1, hidden), lambda i: (i, 0, 0)),
            pl.BlockSpec((1, dim, hidden), lambda i: (i, 0, 0)),
            pl.BlockSpec((1, 1, dim), lambda i: (i, 0, 0)),
            pl.BlockSpec((1, cap_pad, 1), lambda i: (i, 0, 0)),
        ],
        out_specs=pl.BlockSpec((1, cap_pad, dim), lambda i: (i, 0, 0)),
        out_shape=jax.ShapeDtypeStruct((e, cap_pad, dim), jnp.float32),
    )(x.reshape(e, cap_pad, dim), w1, b1.reshape(e, 1, hidden),
      w2, b2.reshape(e, 1, dim), wgt_buf.reshape(e, cap_pad, 1))


def _make_combine_kernel(n, dim, n_workers):
    tok_per_w = n // n_workers
    chunk = 16
    n_chunks = tok_per_w // chunk
    nc = 2
    mesh = plsc.VectorSubcoreMesh(core_axis_name="c", subcore_axis_name="s")

    @functools.partial(
        pl.kernel, mesh=mesh,
        out_type=jax.ShapeDtypeStruct((n, dim), jnp.float32),
        scratch_types=[
            pltpu.VMEM((chunk,), jnp.int32),
            pltpu.VMEM((chunk,), jnp.int32),
            pltpu.VMEM((chunk,), jnp.int32),
            pltpu.VMEM((chunk,), jnp.int32),
            pltpu.VMEM((chunk, dim), jnp.float32),
            pltpu.VMEM((chunk, dim), jnp.float32),
            pltpu.VMEM((chunk, dim), jnp.float32),
            pltpu.VMEM((chunk, dim), jnp.float32),
            pltpu.VMEM((chunk, dim), jnp.float32),
            pltpu.VMEM((chunk, dim), jnp.float32),
            pltpu.SemaphoreType.DMA,
            pltpu.SemaphoreType.DMA,
            pltpu.SemaphoreType.DMA,
            pltpu.SemaphoreType.DMA,
            pltpu.SemaphoreType.DMA,
            pltpu.SemaphoreType.DMA,
        ],
    )
    def combine_kernel(y_hbm, g0_hbm, g1_hbm, out_hbm,
                       i00_v, i01_v, i10_v, i11_v, a0_v, a1_v, b0_v, b1_v,
                       o0_v, o1_v, sem_a0, sem_a1, sem_b0, sem_b1,
                       sem_o0, sem_o1):
        wid = lax.axis_index("s") * nc + lax.axis_index("c")
        tbase = wid * tok_per_w
        ia = (i00_v, i01_v)
        ib = (i10_v, i11_v)
        abuf = (a0_v, a1_v)
        bbuf = (b0_v, b1_v)
        obuf = (o0_v, o1_v)
        sa = (sem_a0, sem_a1)
        sb = (sem_b0, sem_b1)
        so = (sem_o0, sem_o1)

        def issue(c):
            p = c % 2
            t0 = tbase + c * chunk
            pltpu.sync_copy(g0_hbm.at[pl.ds(t0, chunk)], ia[p])
            pltpu.sync_copy(g1_hbm.at[pl.ds(t0, chunk)], ib[p])
            return (pltpu.async_copy(y_hbm.at[ia[p]], abuf[p], sa[p]),
                    pltpu.async_copy(y_hbm.at[ib[p]], bbuf[p], sb[p]))

        gath = issue(0)
        w_prev = None
        for c in range(n_chunks):
            p = c % 2
            cp_a, cp_b = gath
            cp_a.wait()
            cp_b.wait()
            if w_prev is not None:
                w_prev.wait()
            if c + 1 < n_chunks:
                gath = issue(c + 1)
            a_v, b_v, o_v = abuf[p], bbuf[p], obuf[p]

            def add_row(t, _):
                for k in range(dim // 16):
                    s = pl.ds(k * 16, 16)
                    o_v[t, s] = a_v[t, s] + b_v[t, s]
                return 0

            lax.fori_loop(0, chunk, add_row, 0)
            w_prev = pltpu.async_copy(
                o_v, out_hbm.at[pl.ds(tbase + c * chunk, chunk)], so[p])
        w_prev.wait()

    return combine_kernel


def kernel(context, gate_w, gate_b, w1, b1, w2, b2):
    n, dim = context.shape
    e, hidden, _ = w1.shape
    cap = max(1, math.ceil(n * _TOPK / float(e) * _CAPF))
    cap_pad = cap + 1
    rows_total = e * cap_pad
    sent = rows_total - 1
    s_total = n * _TOPK
    n_workers = 32

    i1, i2, s1, s2, imp_sum, ent_sum, cnt = _gating(context, gate_w, gate_b)
    i1 = i1.reshape(n)
    i2 = i2.reshape(n)
    s1 = s1.reshape(n)
    s2 = s2.reshape(n)
    e_flat = jnp.stack([i1, i2], axis=-1).reshape(-1)
    s_flat = jnp.stack([s1, s2], axis=-1).reshape(-1)

    dst, wgt_flat = _rank(e_flat, s_flat, cap, cap_pad, sent)
    dst = dst.reshape(-1)
    wgt_flat = wgt_flat.reshape(-1)

    x_buf, wgt_buf = _make_dispatch_kernel(s_total, rows_total, dim, n_workers)(
        dst, wgt_flat, context)

    y = _ffn(x_buf, w1, b1, w2, b2, wgt_buf, cap_pad)
    y = y.reshape(rows_total, dim)

    g = dst.reshape(n, _TOPK)
    output = _make_combine_kernel(n, dim, n_workers)(
        y, g[:, 0], g[:, 1])

    dispatch = jnp.minimum(cnt.reshape(e), float(cap))
    load = dispatch / jnp.maximum(dispatch.sum(), 1.0)
    importance = imp_sum.reshape(e) / n
    aux_loss = (importance * load).sum() * e
    entropy = ent_sum.reshape(()) / n
    return output, aux_loss, entropy


# speedup vs baseline: 5.4635x; 1.0011x over previous
import functools
import math

import jax
import jax.numpy as jnp
from jax import lax
from jax.experimental import pallas as pl
from jax.experimental.pallas import tpu as pltpu
from jax.experimental.pallas import tpu_sc as plsc

_E = 64
_TOPK = 2
_TEMP = 1.0
_CAPF = 1.1

_TB = 256
_RB = 128


def _gating_body(x_ref, gw_ref, gb_ref,
                 i1_ref, i2_ref, s1_ref, s2_ref, imp_ref, ent_ref, cnt_ref):
    pid = pl.program_id(0)
    x = x_ref[...]
    gw = gw_ref[...]
    gb = gb_ref[...]
    logits = lax.dot_general(x, gw, (((1,), (1,)), ((), ())),
                             preferred_element_type=jnp.float32) + gb
    scaled = logits / _TEMP
    m = jnp.max(scaled, axis=1, keepdims=True)
    p = jnp.exp(scaled - m)
    probs = p / jnp.sum(p, axis=1, keepdims=True)
    ent = -jnp.sum(probs * jnp.log(jnp.clip(probs, 1e-9, None)))
    cols = lax.broadcasted_iota(jnp.int32, scaled.shape, 1)
    v1 = jnp.max(scaled, axis=1)
    i1 = jnp.min(jnp.where(scaled == v1[:, None], cols, _E), axis=1)
    masked = jnp.where(cols == i1[:, None], -jnp.inf, scaled)
    v2 = jnp.max(masked, axis=1)
    i2 = jnp.min(jnp.where(masked == v2[:, None], cols, _E), axis=1)
    t = jnp.exp(v2 - v1)
    s1 = 1.0 / (1.0 + t)
    s2 = t / (1.0 + t)
    i1_ref[0, 0, :] = i1
    i2_ref[0, 0, :] = i2
    s1_ref[0, 0, :] = s1
    s2_ref[0, 0, :] = s2

    oh = ((cols == i1[:, None]).astype(jnp.float32)
          + (cols == i2[:, None]).astype(jnp.float32))

    @pl.when(pid == 0)
    def _():
        imp_ref[...] = jnp.zeros_like(imp_ref)
        ent_ref[...] = jnp.zeros_like(ent_ref)
        cnt_ref[...] = jnp.zeros_like(cnt_ref)

    imp_ref[...] += jnp.sum(probs, axis=0, keepdims=True)
    ent_ref[...] += jnp.full((1, 1), 0.0) + ent
    cnt_ref[...] += jnp.sum(oh, axis=0, keepdims=True)


def _gating(context, gate_w, gate_b):
    n, dim = context.shape
    nb = n // _TB
    out_shapes = [
        jax.ShapeDtypeStruct((nb, 1, _TB), jnp.int32),
        jax.ShapeDtypeStruct((nb, 1, _TB), jnp.int32),
        jax.ShapeDtypeStruct((nb, 1, _TB), jnp.float32),
        jax.ShapeDtypeStruct((nb, 1, _TB), jnp.float32),
        jax.ShapeDtypeStruct((1, _E), jnp.float32),
        jax.ShapeDtypeStruct((1, 1), jnp.float32),
        jax.ShapeDtypeStruct((1, _E), jnp.float32),
    ]
    tok_spec = pl.BlockSpec((1, 1, _TB), lambda i: (i, 0, 0))
    acc_e = pl.BlockSpec((1, _E), lambda i: (0, 0))
    return pl.pallas_call(
        _gating_body,
        grid=(nb,),
        in_specs=[
            pl.BlockSpec((_TB, dim), lambda i: (i, 0)),
            pl.BlockSpec((_E, dim), lambda i: (0, 0)),
            pl.BlockSpec((1, _E), lambda i: (0, 0)),
        ],
        out_specs=[tok_spec, tok_spec, tok_spec, tok_spec,
                   acc_e, pl.BlockSpec((1, 1), lambda i: (0, 0)), acc_e],
        out_shape=out_shapes,
    )(context, gate_w, gate_b.reshape(1, _E))


def _rank_body(cap, cap_pad, sent, ec_ref, sc_ref, er_ref, sr_ref,
               dst_ref, wgt_ref):
    b = pl.program_id(0)
    ec = ec_ref[...]
    sc = sc_ref[...]
    er = er_ref[...]
    sr = sr_ref[...]
    s_total = er.shape[1]
    i_idx = b * _RB + lax.broadcasted_iota(jnp.int32, (_RB, 1), 0)
    j_idx = lax.broadcasted_iota(jnp.int32, (1, s_total), 1)
    same = er == ec
    better = (sr > sc) | ((sr == sc) & (j_idx < i_idx))
    rank = jnp.sum((same & better).astype(jnp.int32), axis=1, keepdims=True)
    keep = rank < cap
    dst_ref[...] = jnp.where(keep, ec * cap_pad + rank, sent)
    wgt_ref[...] = jnp.where(keep, sc, 0.0)


def _rank(e_flat, s_flat, cap, cap_pad, sent):
    s_total = e_flat.shape[0]
    nb = s_total // _RB
    return pl.pallas_call(
        functools.partial(_rank_body, cap, cap_pad, sent),
        grid=(nb,),
        compiler_params=pltpu.CompilerParams(
            dimension_semantics=("parallel",)),
        in_specs=[
            pl.BlockSpec((_RB, 1), lambda i: (i, 0)),
            pl.BlockSpec((_RB, 1), lambda i: (i, 0)),
            pl.BlockSpec((1, s_total), lambda i: (0, 0)),
            pl.BlockSpec((1, s_total), lambda i: (0, 0)),
        ],
        out_specs=[pl.BlockSpec((_RB, 1), lambda i: (i, 0)),
                   pl.BlockSpec((_RB, 1), lambda i: (i, 0))],
        out_shape=[jax.ShapeDtypeStruct((s_total, 1), jnp.int32),
                   jax.ShapeDtypeStruct((s_total, 1), jnp.float32)],
    )(e_flat.reshape(s_total, 1), s_flat.reshape(s_total, 1),
      e_flat.reshape(1, s_total), s_flat.reshape(1, s_total))


def _make_dispatch_kernel(s_total, rows_total, dim, n_workers):
    rows_per_w = rows_total // n_workers
    chunk = 48
    n_chunks = rows_per_w // chunk
    nc = 2
    mesh = plsc.VectorSubcoreMesh(core_axis_name="c", subcore_axis_name="s")

    @functools.partial(
        pl.kernel, mesh=mesh,
        compiler_params=pltpu.CompilerParams(needs_layout_passes=False),
        out_type=[jax.ShapeDtypeStruct((rows_total, dim), jnp.float32),
                  jax.ShapeDtypeStruct((rows_total,), jnp.float32)],
        scratch_types=[
            pltpu.VMEM((s_total,), jnp.int32),
            pltpu.VMEM((s_total,), jnp.float32),
            pltpu.VMEM((rows_per_w,), jnp.int32),
            pltpu.VMEM((rows_per_w,), jnp.float32),
            pltpu.VMEM((chunk,), jnp.int32),
            pltpu.VMEM((chunk,), jnp.int32),
            pltpu.VMEM((chunk, dim), jnp.float32),
            pltpu.VMEM((chunk, dim), jnp.float32),
            pltpu.SemaphoreType.DMA,
            pltpu.SemaphoreType.DMA,
            pltpu.SemaphoreType.DMA,
            pltpu.SemaphoreType.DMA,
        ],
    )
    def dispatch_kernel(dst_hbm, wgtf_hbm, ctx_hbm, x_hbm, wgtb_hbm,
                        dst_v, wgtf_v, tok_v, wgtb_v, idx0_v, idx1_v,
                        rows0_v, rows1_v, sem_g0, sem_g1, sem_w0, sem_w1):
        wid = lax.axis_index("s") * nc + lax.axis_index("c")
        base = wid * rows_per_w
        pltpu.sync_copy(dst_hbm, dst_v)
        pltpu.sync_copy(wgtf_hbm, wgtf_v)
        zero = wid * 0
        zi = lax.broadcast(zero, (16,))
        zf = lax.broadcast(zero.astype(jnp.float32), (16,))
        for k in range(rows_per_w // 16):
            tok_v[pl.ds(k * 16, 16)] = zi
            wgtb_v[pl.ds(k * 16, 16)] = zf

        base_iota = lax.iota(jnp.int32, 16)
        unroll = 4

        def scan_body(i, _):
            for u in range(unroll):
                off = i * (16 * unroll) + u * 16
                d16 = dst_v[pl.ds(off, 16)]
                w16 = wgtf_v[pl.ds(off, 16)]
                slot = off + base_iota
                tok16 = lax.shift_right_logical(slot, 1)
                msk = (d16 >= base) & (d16 < base + rows_per_w)
                loc = jnp.where(msk, d16 - base, 0)
                plsc.store_scatter(tok_v, [loc], tok16, mask=msk)
                plsc.store_scatter(wgtb_v, [loc], w16, mask=msk)
            return 0

        lax.fori_loop(0, s_total // (16 * unroll), scan_body, 0)
        pltpu.async_copy(wgtb_v, wgtb_hbm.at[pl.ds(base, rows_per_w)], sem_w0).wait()

        idx_bufs = (idx0_v, idx1_v)
        row_bufs = (rows0_v, rows1_v)
        g_sems = (sem_g0, sem_g1)
        w_sems = (sem_w0, sem_w1)

        def issue_gather(c):
            idx_v = idx_bufs[c % 2]
            for k in range(chunk // 16):
                idx_v[pl.ds(k * 16, 16)] = tok_v[pl.ds(c * chunk + k * 16, 16)]
            return pltpu.async_copy(ctx_hbm.at[idx_v], row_bufs[c % 2],
                                    g_sems[c % 2])

        g = issue_gather(0)
        w_prev = None
        for c in range(n_chunks):
            g.wait()
            if w_prev is not None:
                w_prev.wait()
            if c + 1 < n_chunks:
                g = issue_gather(c + 1)
            w_prev = pltpu.async_copy(
                row_bufs[c % 2], x_hbm.at[pl.ds(base + c * chunk, chunk)],
                w_sems[c % 2])
        w_prev.wait()

    return dispatch_kernel


def _ffn_body(x_ref, w1_ref, b1_ref, w2_ref, b2_ref, wgt_ref, y_ref):
    x = x_ref[0]
    w1 = w1_ref[0]
    w2 = w2_ref[0]
    h = lax.dot_general(x, w1, (((1,), (1,)), ((), ())),
                        preferred_element_type=jnp.float32) + b1_ref[0]
    h = jnp.maximum(h, 0.0)
    y = lax.dot_general(h, w2, (((1,), (1,)), ((), ())),
                        preferred_element_type=jnp.float32) + b2_ref[0]
    y_ref[0] = y * wgt_ref[0]


def _ffn(x, w1, b1, w2, b2, wgt_buf, cap_pad):
    e, hidden, dim = w1.shape
    return pl.pallas_call(
        _ffn_body,
        grid=(e,),
        compiler_params=pltpu.CompilerParams(
            dimension_semantics=("parallel",)),
        in_specs=[
            pl.BlockSpec((1, cap_pad, dim), lambda i: (i, 0, 0)),
            pl.BlockSpec((1, hidden, dim), lambda i: (i, 0, 0)),
            pl.BlockSpec((1, 1, hidden), lambda i: (i, 0, 0)),
            pl.BlockSpec((1, dim, hidden), lambda i: (i, 0, 0)),
            pl.BlockSpec((1, 1, dim), lambda i: (i, 0, 0)),
            pl.BlockSpec((1, cap_pad, 1), lambda i: (i, 0, 0)),
        ],
        out_specs=pl.BlockSpec((1, cap_pad, dim), lambda i: (i, 0, 0)),
        out_shape=jax.ShapeDtypeStruct((e, cap_pad, dim), jnp.float32),
    )(x.reshape(e, cap_pad, dim), w1, b1.reshape(e, 1, hidden),
      w2, b2.reshape(e, 1, dim), wgt_buf.reshape(e, cap_pad, 1))


def _make_combine_kernel(n, dim, n_workers):
    tok_per_w = n // n_workers
    chunk = 16
    n_chunks = tok_per_w // chunk
    nc = 2
    mesh = plsc.VectorSubcoreMesh(core_axis_name="c", subcore_axis_name="s")

    @functools.partial(
        pl.kernel, mesh=mesh,
        out_type=jax.ShapeDtypeStruct((n, dim), jnp.float32),
        scratch_types=[
            pltpu.VMEM((chunk,), jnp.int32),
            pltpu.VMEM((chunk,), jnp.int32),
            pltpu.VMEM((chunk,), jnp.int32),
            pltpu.VMEM((chunk,), jnp.int32),
            pltpu.VMEM((chunk, dim), jnp.float32),
            pltpu.VMEM((chunk, dim), jnp.float32),
            pltpu.VMEM((chunk, dim), jnp.float32),
            pltpu.VMEM((chunk, dim), jnp.float32),
            pltpu.VMEM((chunk, dim), jnp.float32),
            pltpu.VMEM((chunk, dim), jnp.float32),
            pltpu.SemaphoreType.DMA,
            pltpu.SemaphoreType.DMA,
            pltpu.SemaphoreType.DMA,
            pltpu.SemaphoreType.DMA,
            pltpu.SemaphoreType.DMA,
            pltpu.SemaphoreType.DMA,
        ],
    )
    def combine_kernel(y_hbm, g0_hbm, g1_hbm, out_hbm,
                       i00_v, i01_v, i10_v, i11_v, a0_v, a1_v, b0_v, b1_v,
                       o0_v, o1_v, sem_a0, sem_a1, sem_b0, sem_b1,
                       sem_o0, sem_o1):
        wid = lax.axis_index("s") * nc + lax.axis_index("c")
        tbase = wid * tok_per_w
        ia = (i00_v, i01_v)
        ib = (i10_v, i11_v)
        abuf = (a0_v, a1_v)
        bbuf = (b0_v, b1_v)
        obuf = (o0_v, o1_v)
        sa = (sem_a0, sem_a1)
        sb = (sem_b0, sem_b1)
        so = (sem_o0, sem_o1)

        def issue(c):
            p = c % 2
            t0 = tbase + c * chunk
            pltpu.sync_copy(g0_hbm.at[pl.ds(t0, chunk)], ia[p])
            pltpu.sync_copy(g1_hbm.at[pl.ds(t0, chunk)], ib[p])
            return (pltpu.async_copy(y_hbm.at[ia[p]], abuf[p], sa[p]),
                    pltpu.async_copy(y_hbm.at[ib[p]], bbuf[p], sb[p]))

        gath = issue(0)
        w_prev = None
        for c in range(n_chunks):
            p = c % 2
            cp_a, cp_b = gath
            cp_a.wait()
            cp_b.wait()
            if w_prev is not None:
                w_prev.wait()
            if c + 1 < n_chunks:
                gath = issue(c + 1)
            a_v, b_v, o_v = abuf[p], bbuf[p], obuf[p]

            def add_row(t, _):
                for k in range(dim // 16):
                    s = pl.ds(k * 16, 16)
                    o_v[t, s] = a_v[t, s] + b_v[t, s]
                return 0

            lax.fori_loop(0, chunk, add_row, 0)
            w_prev = pltpu.async_copy(
                o_v, out_hbm.at[pl.ds(tbase + c * chunk, chunk)], so[p])
        w_prev.wait()

    return combine_kernel


def kernel(context, gate_w, gate_b, w1, b1, w2, b2):
    n, dim = context.shape
    e, hidden, _ = w1.shape
    cap = max(1, math.ceil(n * _TOPK / float(e) * _CAPF))
    cap_pad = cap + 1
    rows_total = e * cap_pad
    sent = rows_total - 1
    s_total = n * _TOPK
    n_workers = 32

    i1, i2, s1, s2, imp_sum, ent_sum, cnt = _gating(context, gate_w, gate_b)
    i1 = i1.reshape(n)
    i2 = i2.reshape(n)
    s1 = s1.reshape(n)
    s2 = s2.reshape(n)
    e_flat = jnp.stack([i1, i2], axis=-1).reshape(-1)
    s_flat = jnp.stack([s1, s2], axis=-1).reshape(-1)

    dst, wgt_flat = _rank(e_flat, s_flat, cap, cap_pad, sent)
    dst = dst.reshape(-1)
    wgt_flat = wgt_flat.reshape(-1)

    x_buf, wgt_buf = _make_dispatch_kernel(s_total, rows_total, dim, n_workers)(
        dst, wgt_flat, context)

    y = _ffn(x_buf, w1, b1, w2, b2, wgt_buf, cap_pad)
    y = y.reshape(rows_total, dim)

    g = dst.reshape(n, _TOPK)
    output = _make_combine_kernel(n, dim, n_workers)(
        y, g[:, 0], g[:, 1])

    dispatch = jnp.minimum(cnt.reshape(e), float(cap))
    load = dispatch / jnp.maximum(dispatch.sum(), 1.0)
    importance = imp_sum.reshape(e) / n
    aux_loss = (importance * load).sum() * e
    entropy = ent_sum.reshape(()) / n
    return output, aux_loss, entropy


# speedup vs baseline: 6.2792x; 1.1493x over previous
import functools
import math

import jax
import jax.numpy as jnp
from jax import lax
from jax.experimental import pallas as pl
from jax.experimental.pallas import tpu as pltpu
from jax.experimental.pallas import tpu_sc as plsc

_E = 64
_TOPK = 2
_TEMP = 1.0
_CAPF = 1.1

_TB = 256
_RB = 128


def _gating_body(x_ref, gw_ref, gb_ref,
                 i1_ref, i2_ref, s1_ref, s2_ref, imp_ref, ent_ref, cnt_ref):
    pid = pl.program_id(0)
    x = x_ref[...]
    gw = gw_ref[...]
    gb = gb_ref[...]
    logits = lax.dot_general(x, gw, (((1,), (1,)), ((), ())),
                             preferred_element_type=jnp.float32) + gb
    scaled = logits / _TEMP
    m = jnp.max(scaled, axis=1, keepdims=True)
    p = jnp.exp(scaled - m)
    probs = p / jnp.sum(p, axis=1, keepdims=True)
    ent = -jnp.sum(probs * jnp.log(jnp.clip(probs, 1e-9, None)))
    cols = lax.broadcasted_iota(jnp.int32, scaled.shape, 1)
    v1 = jnp.max(scaled, axis=1)
    i1 = jnp.min(jnp.where(scaled == v1[:, None], cols, _E), axis=1)
    masked = jnp.where(cols == i1[:, None], -jnp.inf, scaled)
    v2 = jnp.max(masked, axis=1)
    i2 = jnp.min(jnp.where(masked == v2[:, None], cols, _E), axis=1)
    t = jnp.exp(v2 - v1)
    s1 = 1.0 / (1.0 + t)
    s2 = t / (1.0 + t)
    i1_ref[0, 0, :] = i1
    i2_ref[0, 0, :] = i2
    s1_ref[0, 0, :] = s1
    s2_ref[0, 0, :] = s2

    oh = ((cols == i1[:, None]).astype(jnp.float32)
          + (cols == i2[:, None]).astype(jnp.float32))

    @pl.when(pid == 0)
    def _():
        imp_ref[...] = jnp.zeros_like(imp_ref)
        ent_ref[...] = jnp.zeros_like(ent_ref)
        cnt_ref[...] = jnp.zeros_like(cnt_ref)

    imp_ref[...] += jnp.sum(probs, axis=0, keepdims=True)
    ent_ref[...] += jnp.full((1, 1), 0.0) + ent
    cnt_ref[...] += jnp.sum(oh, axis=0, keepdims=True)


def _gating(context, gate_w, gate_b):
    n, dim = context.shape
    nb = n // _TB
    out_shapes = [
        jax.ShapeDtypeStruct((nb, 1, _TB), jnp.int32),
        jax.ShapeDtypeStruct((nb, 1, _TB), jnp.int32),
        jax.ShapeDtypeStruct((nb, 1, _TB), jnp.float32),
        jax.ShapeDtypeStruct((nb, 1, _TB), jnp.float32),
        jax.ShapeDtypeStruct((1, _E), jnp.float32),
        jax.ShapeDtypeStruct((1, 1), jnp.float32),
        jax.ShapeDtypeStruct((1, _E), jnp.float32),
    ]
    tok_spec = pl.BlockSpec((1, 1, _TB), lambda i: (i, 0, 0))
    acc_e = pl.BlockSpec((1, _E), lambda i: (0, 0))
    return pl.pallas_call(
        _gating_body,
        grid=(nb,),
        in_specs=[
            pl.BlockSpec((_TB, dim), lambda i: (i, 0)),
            pl.BlockSpec((_E, dim), lambda i: (0, 0)),
            pl.BlockSpec((1, _E), lambda i: (0, 0)),
        ],
        out_specs=[tok_spec, tok_spec, tok_spec, tok_spec,
                   acc_e, pl.BlockSpec((1, 1), lambda i: (0, 0)), acc_e],
        out_shape=out_shapes,
    )(context, gate_w, gate_b.reshape(1, _E))


def _rank_body(cap, cap_pad, sent, ec_ref, sc_ref, er_ref, sr_ref,
               dst_ref, wgt_ref):
    b = pl.program_id(0)
    ec = ec_ref[...]
    sc = sc_ref[...]
    er = er_ref[...]
    sr = sr_ref[...]
    s_total = er.shape[1]
    i_idx = b * _RB + lax.broadcasted_iota(jnp.int32, (_RB, 1), 0)
    j_idx = lax.broadcasted_iota(jnp.int32, (1, s_total), 1)
    same = er == ec
    better = (sr > sc) | ((sr == sc) & (j_idx < i_idx))
    rank = jnp.sum((same & better).astype(jnp.int32), axis=1, keepdims=True)
    keep = rank < cap
    dst_ref[...] = jnp.where(keep, ec * cap_pad + rank, sent)
    wgt_ref[...] = jnp.where(keep, sc, 0.0)


def _rank(e_flat, s_flat, cap, cap_pad, sent):
    s_total = e_flat.shape[0]
    nb = s_total // _RB
    return pl.pallas_call(
        functools.partial(_rank_body, cap, cap_pad, sent),
        grid=(nb,),
        compiler_params=pltpu.CompilerParams(
            dimension_semantics=("parallel",)),
        in_specs=[
            pl.BlockSpec((_RB, 1), lambda i: (i, 0)),
            pl.BlockSpec((_RB, 1), lambda i: (i, 0)),
            pl.BlockSpec((1, s_total), lambda i: (0, 0)),
            pl.BlockSpec((1, s_total), lambda i: (0, 0)),
        ],
        out_specs=[pl.BlockSpec((_RB, 1), lambda i: (i, 0)),
                   pl.BlockSpec((_RB, 1), lambda i: (i, 0))],
        out_shape=[jax.ShapeDtypeStruct((s_total, 1), jnp.int32),
                   jax.ShapeDtypeStruct((s_total, 1), jnp.float32)],
    )(e_flat.reshape(s_total, 1), s_flat.reshape(s_total, 1),
      e_flat.reshape(1, s_total), s_flat.reshape(1, s_total))


def _make_dispatch_kernel(s_total, rows_total, dim, n_workers):
    rows_per_w = rows_total // n_workers
    chunk = 48
    n_chunks = rows_per_w // chunk
    nc = 2
    mesh = plsc.VectorSubcoreMesh(core_axis_name="c", subcore_axis_name="s")

    @functools.partial(
        pl.kernel, mesh=mesh,
        compiler_params=pltpu.CompilerParams(needs_layout_passes=False),
        out_type=[jax.ShapeDtypeStruct((rows_total, dim), jnp.float32),
                  jax.ShapeDtypeStruct((rows_total,), jnp.float32)],
        scratch_types=[
            pltpu.VMEM((s_total,), jnp.int32),
            pltpu.VMEM((s_total,), jnp.float32),
            pltpu.VMEM((rows_per_w,), jnp.int32),
            pltpu.VMEM((rows_per_w,), jnp.float32),
            pltpu.VMEM((chunk,), jnp.int32),
            pltpu.VMEM((chunk,), jnp.int32),
            pltpu.VMEM((chunk, dim), jnp.float32),
            pltpu.VMEM((chunk, dim), jnp.float32),
            pltpu.SemaphoreType.DMA,
            pltpu.SemaphoreType.DMA,
            pltpu.SemaphoreType.DMA,
            pltpu.SemaphoreType.DMA,
        ],
    )
    def dispatch_kernel(dst_hbm, wgtf_hbm, ctx_hbm, x_hbm, wgtb_hbm,
                        dst_v, wgtf_v, tok_v, wgtb_v, idx0_v, idx1_v,
                        rows0_v, rows1_v, sem_g0, sem_g1, sem_w0, sem_w1):
        wid = lax.axis_index("s") * nc + lax.axis_index("c")
        base = wid * rows_per_w
        pltpu.sync_copy(dst_hbm, dst_v)
        pltpu.sync_copy(wgtf_hbm, wgtf_v)
        zero = wid * 0
        zi = lax.broadcast(zero, (16,))
        zf = lax.broadcast(zero.astype(jnp.float32), (16,))
        for k in range(rows_per_w // 16):
            tok_v[pl.ds(k * 16, 16)] = zi
            wgtb_v[pl.ds(k * 16, 16)] = zf

        base_iota = lax.iota(jnp.int32, 16)
        unroll = 4

        def scan_body(i, _):
            for u in range(unroll):
                off = i * (16 * unroll) + u * 16
                d16 = dst_v[pl.ds(off, 16)]
                w16 = wgtf_v[pl.ds(off, 16)]
                slot = off + base_iota
                tok16 = lax.shift_right_logical(slot, 1)
                msk = (d16 >= base) & (d16 < base + rows_per_w)
                loc = jnp.where(msk, d16 - base, 0)
                plsc.store_scatter(tok_v, [loc], tok16, mask=msk)
                plsc.store_scatter(wgtb_v, [loc], w16, mask=msk)
            return 0

        lax.fori_loop(0, s_total // (16 * unroll), scan_body, 0)
        pltpu.async_copy(wgtb_v, wgtb_hbm.at[pl.ds(base, rows_per_w)], sem_w0).wait()

        idx_bufs = (idx0_v, idx1_v)
        row_bufs = (rows0_v, rows1_v)
        g_sems = (sem_g0, sem_g1)
        w_sems = (sem_w0, sem_w1)

        def issue_gather(c):
            idx_v = idx_bufs[c % 2]
            for k in range(chunk // 16):
                idx_v[pl.ds(k * 16, 16)] = tok_v[pl.ds(c * chunk + k * 16, 16)]
            return pltpu.async_copy(ctx_hbm.at[idx_v], row_bufs[c % 2],
                                    g_sems[c % 2])

        g = issue_gather(0)
        w_prev = None
        for c in range(n_chunks):
            g.wait()
            if w_prev is not None:
                w_prev.wait()
            if c + 1 < n_chunks:
                g = issue_gather(c + 1)
            w_prev = pltpu.async_copy(
                row_bufs[c % 2], x_hbm.at[pl.ds(base + c * chunk, chunk)],
                w_sems[c % 2])
        w_prev.wait()

    return dispatch_kernel


_EB = 4


def _ffn_body(x_ref, w1_ref, b1_ref, w2_ref, b2_ref, wgt_ref, y_ref):
    for k in range(_EB):
        x = x_ref[k]
        w1 = w1_ref[k]
        w2 = w2_ref[k]
        h = lax.dot_general(x, w1, (((1,), (1,)), ((), ())),
                            preferred_element_type=jnp.float32) + b1_ref[k]
        h = jnp.maximum(h, 0.0)
        y = lax.dot_general(h, w2, (((1,), (1,)), ((), ())),
                            preferred_element_type=jnp.float32) + b2_ref[k]
        y_ref[k] = y * wgt_ref[k]


def _ffn(x, w1, b1, w2, b2, wgt_buf, cap_pad):
    e, hidden, dim = w1.shape
    return pl.pallas_call(
        _ffn_body,
        grid=(e // _EB,),
        compiler_params=pltpu.CompilerParams(
            dimension_semantics=("parallel",)),
        in_specs=[
            pl.BlockSpec((_EB, cap_pad, dim), lambda i: (i, 0, 0)),
            pl.BlockSpec((_EB, hidden, dim), lambda i: (i, 0, 0)),
            pl.BlockSpec((_EB, 1, hidden), lambda i: (i, 0, 0)),
            pl.BlockSpec((_EB, dim, hidden), lambda i: (i, 0, 0)),
            pl.BlockSpec((_EB, 1, dim), lambda i: (i, 0, 0)),
            pl.BlockSpec((_EB, cap_pad, 1), lambda i: (i, 0, 0)),
        ],
        out_specs=pl.BlockSpec((_EB, cap_pad, dim), lambda i: (i, 0, 0)),
        out_shape=jax.ShapeDtypeStruct((e, cap_pad, dim), jnp.float32),
    )(x.reshape(e, cap_pad, dim), w1, b1.reshape(e, 1, hidden),
      w2, b2.reshape(e, 1, dim), wgt_buf.reshape(e, cap_pad, 1))


def _make_combine_kernel(n, dim, n_workers):
    tok_per_w = n // n_workers
    chunk = 16
    n_chunks = tok_per_w // chunk
    nc = 2
    mesh = plsc.VectorSubcoreMesh(core_axis_name="c", subcore_axis_name="s")

    @functools.partial(
        pl.kernel, mesh=mesh,
        out_type=jax.ShapeDtypeStruct((n, dim), jnp.float32),
        scratch_types=[
            pltpu.VMEM((chunk,), jnp.int32),
            pltpu.VMEM((chunk,), jnp.int32),
            pltpu.VMEM((chunk,), jnp.int32),
            pltpu.VMEM((chunk,), jnp.int32),
            pltpu.VMEM((chunk, dim), jnp.float32),
            pltpu.VMEM((chunk, dim), jnp.float32),
            pltpu.VMEM((chunk, dim), jnp.float32),
            pltpu.VMEM((chunk, dim), jnp.float32),
            pltpu.VMEM((chunk, dim), jnp.float32),
            pltpu.VMEM((chunk, dim), jnp.float32),
            pltpu.SemaphoreType.DMA,
            pltpu.SemaphoreType.DMA,
            pltpu.SemaphoreType.DMA,
            pltpu.SemaphoreType.DMA,
            pltpu.SemaphoreType.DMA,
            pltpu.SemaphoreType.DMA,
        ],
    )
    def combine_kernel(y_hbm, g0_hbm, g1_hbm, out_hbm,
                       i00_v, i01_v, i10_v, i11_v, a0_v, a1_v, b0_v, b1_v,
                       o0_v, o1_v, sem_a0, sem_a1, sem_b0, sem_b1,
                       sem_o0, sem_o1):
        wid = lax.axis_index("s") * nc + lax.axis_index("c")
        tbase = wid * tok_per_w
        ia = (i00_v, i01_v)
        ib = (i10_v, i11_v)
        abuf = (a0_v, a1_v)
        bbuf = (b0_v, b1_v)
        obuf = (o0_v, o1_v)
        sa = (sem_a0, sem_a1)
        sb = (sem_b0, sem_b1)
        so = (sem_o0, sem_o1)

        def issue(c):
            p = c % 2
            t0 = tbase + c * chunk
            pltpu.sync_copy(g0_hbm.at[pl.ds(t0, chunk)], ia[p])
            pltpu.sync_copy(g1_hbm.at[pl.ds(t0, chunk)], ib[p])
            return (pltpu.async_copy(y_hbm.at[ia[p]], abuf[p], sa[p]),
                    pltpu.async_copy(y_hbm.at[ib[p]], bbuf[p], sb[p]))

        gath = issue(0)
        w_prev = None
        for c in range(n_chunks):
            p = c % 2
            cp_a, cp_b = gath
            cp_a.wait()
            cp_b.wait()
            if w_prev is not None:
                w_prev.wait()
            if c + 1 < n_chunks:
                gath = issue(c + 1)
            a_v, b_v, o_v = abuf[p], bbuf[p], obuf[p]

            def add_row(t, _):
                for k in range(dim // 16):
                    s = pl.ds(k * 16, 16)
                    o_v[t, s] = a_v[t, s] + b_v[t, s]
                return 0

            lax.fori_loop(0, chunk, add_row, 0)
            w_prev = pltpu.async_copy(
                o_v, out_hbm.at[pl.ds(tbase + c * chunk, chunk)], so[p])
        w_prev.wait()

    return combine_kernel


def kernel(context, gate_w, gate_b, w1, b1, w2, b2):
    n, dim = context.shape
    e, hidden, _ = w1.shape
    cap = max(1, math.ceil(n * _TOPK / float(e) * _CAPF))
    cap_pad = cap + 1
    rows_total = e * cap_pad
    sent = rows_total - 1
    s_total = n * _TOPK
    n_workers = 32

    i1, i2, s1, s2, imp_sum, ent_sum, cnt = _gating(context, gate_w, gate_b)
    i1 = i1.reshape(n)
    i2 = i2.reshape(n)
    s1 = s1.reshape(n)
    s2 = s2.reshape(n)
    e_flat = jnp.stack([i1, i2], axis=-1).reshape(-1)
    s_flat = jnp.stack([s1, s2], axis=-1).reshape(-1)

    dst, wgt_flat = _rank(e_flat, s_flat, cap, cap_pad, sent)
    dst = dst.reshape(-1)
    wgt_flat = wgt_flat.reshape(-1)

    x_buf, wgt_buf = _make_dispatch_kernel(s_total, rows_total, dim, n_workers)(
        dst, wgt_flat, context)

    y = _ffn(x_buf, w1, b1, w2, b2, wgt_buf, cap_pad)
    y = y.reshape(rows_total, dim)

    g = dst.reshape(n, _TOPK)
    output = _make_combine_kernel(n, dim, n_workers)(
        y, g[:, 0], g[:, 1])

    dispatch = jnp.minimum(cnt.reshape(e), float(cap))
    load = dispatch / jnp.maximum(dispatch.sum(), 1.0)
    importance = imp_sum.reshape(e) / n
    aux_loss = (importance * load).sum() * e
    entropy = ent_sum.reshape(()) / n
    return output, aux_loss, entropy
